# Initial kernel scaffold; baseline (speedup 1.0000x reference)
import jax
import jax.numpy as jnp
from jax.experimental import pallas as pl


def kernel(x, edge_index, batch, W1, b1, W2, b2, W3, b3, Wfc, bfc):
    raise NotImplementedError("write your pallas kernel here")


# speedup vs baseline: 17.8388x; 17.8388x over previous
import functools

import jax
import jax.numpy as jnp
from jax import lax
from jax.experimental import pallas as pl
from jax.experimental.pallas import tpu as pltpu
from jax.experimental.pallas import tpu_sc as plsc

N = 100000
H = 64
G = 64

NC = 2
NS = 16
NW = NC * NS

NP = 100352
ROWS_PER_TILE = 784
EROWS = ROWS_PER_TILE * NW
EPAD = EROWS * 128
DUMMY = 100224

B = 1024
GRID = NP // B
RPW = NP // NS
K = 8


def _sc_body(num_tables, gather, *refs):
    i = 0
    if gather:
        srcR = refs[i]; i += 1
    dstR = refs[i]; i += 1
    tables = refs[i:i + (num_tables if gather else 0)]
    i += len(tables)
    nouts = num_tables
    outs = refs[i:i + nouts]; i += nouts
    acc = refs[i]; i += 1
    if gather:
        src_buf = refs[i]; i += 1
    dst_buf = refs[i]; i += 1
    rows = refs[i]; i += 1
    zbuf = refs[i]; i += 1
    semg = refs[i]; i += 1
    sems = refs[i]; i += 1

    core = lax.axis_index("c")
    sub = lax.axis_index("s")
    w = core * NS + sub
    r0 = sub * RPW
    row0 = w * ROWS_PER_TILE

    zero16 = jnp.zeros((16,), jnp.float32)
    for z in range(64):
        zbuf[z, :] = zero16
    if not gather:
        one16 = jnp.ones((16,), jnp.float32)
        for z in range(128):
            rows[0, z, :] = one16

    def zero_acc():
        def zl(k, carry):
            pltpu.sync_copy(zbuf, acc.at[pl.ds(r0 + k * 64, 64)])
            return carry
        lax.fori_loop(0, RPW // 64, zl, 0)

    def edge_pass(table):
        def group(g_, carry):
            base = row0 + g_ * K
            pltpu.sync_copy(dstR.at[pl.ds(base, K)], dst_buf)
            if gather:
                pltpu.sync_copy(srcR.at[pl.ds(base, K)], src_buf)
                descs = []
                for j in range(K):
                    descs.append(pltpu.async_copy(
                        table.at[src_buf.at[j]], rows.at[j], semg))
                for d in descs:
                    d.wait()
            descs = []
            for j in range(K):
                rsrc = rows.at[j] if gather else rows.at[0]
                descs.append(pltpu.async_copy(
                    rsrc, acc.at[dst_buf.at[j]], sems, add=True))
            for d in descs:
                d.wait()
            return carry
        lax.fori_loop(0, ROWS_PER_TILE // K, group, 0)

    for p in range(num_tables):
        zero_acc()
        plsc.subcore_barrier()
        edge_pass(tables[p] if gather else None)
        plsc.subcore_barrier()
        pltpu.sync_copy(acc.at[pl.ds(r0, RPW)],
                        outs[p].at[core, pl.ds(r0, RPW)])
        if p + 1 < num_tables:
            plsc.subcore_barrier()


def _make_sc(num_tables, gather):
    mesh = plsc.VectorSubcoreMesh(core_axis_name="c", subcore_axis_name="s",
                                  num_cores=NC, num_subcores=NS)
    out_type = [jax.ShapeDtypeStruct((NC, NP, 16), jnp.float32)
                for _ in range(num_tables)]
    scratch = [pltpu.VMEM_SHARED((NP, 16), jnp.float32)]
    if gather:
        scratch.append(pltpu.VMEM((K, 128), jnp.int32))
    scratch += [
        pltpu.VMEM((K, 128), jnp.int32),
        pltpu.VMEM((K, 128, 16), jnp.float32),
        pltpu.VMEM((64, 16), jnp.float32),
        pltpu.SemaphoreType.DMA,
        pltpu.SemaphoreType.DMA,
    ]
    body = functools.partial(_sc_body, num_tables, gather)
    return pl.kernel(body, out_type=out_type, mesh=mesh,
                     scratch_types=scratch,
                     compiler_params=pltpu.CompilerParams(
                         use_tc_tiling_on_sc=False))


_sc_deg = _make_sc(1, gather=False)
_sc_prop1 = _make_sc(1, gather=True)
_sc_prop4 = _make_sc(4, gather=True)


def _rows_mask(pid):
    idx = pid * B + lax.broadcasted_iota(jnp.int32, (B, 1), 0)
    return idx < N


def _tc1_body(deg_ref, x_ref, dinv_ref, xs_ref):
    pid = pl.program_id(0)
    deg = deg_ref[0, :, 0:1] + deg_ref[1, :, 0:1] + 1.0
    dinv = jnp.where(_rows_mask(pid), lax.rsqrt(deg), 0.0)
    dinv_ref[...] = dinv
    xs_ref[...] = jnp.concatenate(
        [dinv * x_ref[...], jnp.zeros((B, 15), jnp.float32)], axis=1)


def _tc1(deg16, xP):
    return pl.pallas_call(
        _tc1_body,
        grid=(GRID,),
        in_specs=[
            pl.BlockSpec((NC, B, 16), lambda i: (0, i, 0)),
            pl.BlockSpec((B, 1), lambda i: (i, 0)),
        ],
        out_specs=[
            pl.BlockSpec((B, 1), lambda i: (i, 0)),
            pl.BlockSpec((B, 16), lambda i: (i, 0)),
        ],
        out_shape=[
            jax.ShapeDtypeStruct((NP, 1), jnp.float32),
            jax.ShapeDtypeStruct((NP, 16), jnp.float32),
        ],
    )(deg16, xP)


def _tc2_body(p16_ref, xs_ref, dinv_ref, w1_ref, b1_ref, *g_refs):
    dinv = dinv_ref[...]
    p = dinv * (p16_ref[0, :, 0:1] + p16_ref[1, :, 0:1] + xs_ref[:, 0:1])
    h = jax.nn.relu(p * w1_ref[...] + b1_ref[...])
    g = dinv * h
    for k in range(4):
        g_refs[k][...] = g[:, 16 * k:16 * (k + 1)]


def _tc2(p16, xs16, dinv, W1, b1):
    return pl.pallas_call(
        _tc2_body,
        grid=(GRID,),
        in_specs=[
            pl.BlockSpec((NC, B, 16), lambda i: (0, i, 0)),
            pl.BlockSpec((B, 16), lambda i: (i, 0)),
            pl.BlockSpec((B, 1), lambda i: (i, 0)),
            pl.BlockSpec((1, H), lambda i: (0, 0)),
            pl.BlockSpec((1, H), lambda i: (0, 0)),
        ],
        out_specs=[pl.BlockSpec((B, 16), lambda i: (i, 0))] * 4,
        out_shape=[jax.ShapeDtypeStruct((NP, 16), jnp.float32)] * 4,
    )(p16, xs16, dinv, W1, b1)


def _tc3_body(a0, a1, a2, a3, g0, g1, g2, g3, dinv_ref, w_ref, b_ref,
              *out_refs):
    dinv = dinv_ref[...]
    accs = (a0, a1, a2, a3)
    gins = (g0, g1, g2, g3)
    z = jnp.concatenate(
        [accs[k][0] + accs[k][1] + gins[k][...] for k in range(4)], axis=1)
    h = jax.nn.relu(jnp.dot(dinv * z, w_ref[...],
                            preferred_element_type=jnp.float32) + b_ref[...])
    g = dinv * h
    for k in range(4):
        out_refs[k][...] = g[:, 16 * k:16 * (k + 1)]


def _tc3(accs, gins, dinv, W, b):
    return pl.pallas_call(
        _tc3_body,
        grid=(GRID,),
        in_specs=(
            [pl.BlockSpec((NC, B, 16), lambda i: (0, i, 0))] * 4
            + [pl.BlockSpec((B, 16), lambda i: (i, 0))] * 4
            + [
                pl.BlockSpec((B, 1), lambda i: (i, 0)),
                pl.BlockSpec((H, H), lambda i: (0, 0)),
                pl.BlockSpec((1, H), lambda i: (0, 0)),
            ]
        ),
        out_specs=[pl.BlockSpec((B, 16), lambda i: (i, 0))] * 4,
        out_shape=[jax.ShapeDtypeStruct((NP, 16), jnp.float32)] * 4,
    )(*accs, *gins, dinv, W, b)


def _tc4_body(a0, a1, a2, a3, g0, g1, g2, g3, dinv_ref, batch_ref, w3_ref,
              b3_ref, wfc_ref, bfc_ref, out_ref, s_acc, c_acc):
    pid = pl.program_id(0)
    dinv = dinv_ref[...]
    accs = (a0, a1, a2, a3)
    gins = (g0, g1, g2, g3)
    z = jnp.concatenate(
        [accs[k][0] + accs[k][1] + gins[k][...] for k in range(4)], axis=1)
    h = jax.nn.relu(jnp.dot(dinv * z, w3_ref[...],
                            preferred_element_type=jnp.float32) + b3_ref[...])
    seg = batch_ref[...]
    segs = lax.broadcasted_iota(jnp.int32, (B, G), 1)
    onehot = jnp.where((seg == segs) & _rows_mask(pid), 1.0, 0.0)

    s_part = lax.dot_general(onehot, h, (((0,), (0,)), ((), ())),
                             preferred_element_type=jnp.float32)
    c_part = lax.dot_general(onehot, jnp.ones((B, 1), jnp.float32),
                             (((0,), (0,)), ((), ())),
                             preferred_element_type=jnp.float32)

    @pl.when(pid == 0)
    def _():
        s_acc[...] = jnp.zeros_like(s_acc)
        c_acc[...] = jnp.zeros_like(c_acc)

    s_acc[...] += s_part
    c_acc[...] += c_part

    @pl.when(pid == GRID - 1)
    def _():
        pooled = s_acc[...] / jnp.maximum(c_acc[...], 1.0)
        out_ref[...] = jnp.dot(pooled, wfc_ref[...],
                               preferred_element_type=jnp.float32) + bfc_ref[...]


def _tc4(accs, gins, dinv, batchP, W3, b3, Wfc, bfc):
    return pl.pallas_call(
        _tc4_body,
        grid=(GRID,),
        in_specs=(
            [pl.BlockSpec((NC, B, 16), lambda i: (0, i, 0))] * 4
            + [pl.BlockSpec((B, 16), lambda i: (i, 0))] * 4
            + [
                pl.BlockSpec((B, 1), lambda i: (i, 0)),
                pl.BlockSpec((B, 1), lambda i: (i, 0)),
                pl.BlockSpec((H, H), lambda i: (0, 0)),
                pl.BlockSpec((1, H), lambda i: (0, 0)),
                pl.BlockSpec((H, 1), lambda i: (0, 0)),
                pl.BlockSpec((1, 1), lambda i: (0, 0)),
            ]
        ),
        out_specs=pl.BlockSpec((G, 1), lambda i: (0, 0)),
        out_shape=jax.ShapeDtypeStruct((G, 1), jnp.float32),
        scratch_shapes=[
            pltpu.VMEM((G, H), jnp.float32),
            pltpu.VMEM((G, 1), jnp.float32),
        ],
    )(*accs, *gins, dinv, batchP, W3, b3, Wfc, bfc)


def kernel(x, edge_index, batch, W1, b1, W2, b2, W3, b3, Wfc, bfc):
    E = edge_index.shape[1]
    pad_e = EPAD - E
    fill = jnp.full((pad_e,), DUMMY, jnp.int32)
    srcR = jnp.concatenate([edge_index[0], fill]).reshape(EROWS, 128)
    dstR = jnp.concatenate([edge_index[1], fill]).reshape(EROWS, 128)

    xP = jnp.pad(x, ((0, NP - N), (0, 0)))
    batchP = jnp.pad(batch, (0, NP - N)).reshape(NP, 1)
    b1r = b1.reshape(1, H)
    b2r = b2.reshape(1, H)
    b3r = b3.reshape(1, H)
    bfcr = bfc.reshape(1, 1)

    (deg16,) = _sc_deg(dstR)
    dinv, xs16 = _tc1(deg16, xP)
    (p16,) = _sc_prop1(srcR, dstR, xs16)
    g1 = _tc2(p16, xs16, dinv, W1, b1r)
    a2 = _sc_prop4(srcR, dstR, *g1)
    g2 = _tc3(a2, g1, dinv, W2, b2r)
    a3 = _sc_prop4(srcR, dstR, *g2)
    return _tc4(a3, g2, dinv, batchP, W3, b3r, Wfc, bfcr)


# speedup vs baseline: 18.7527x; 1.0512x over previous
import functools

import jax
import jax.numpy as jnp
from jax import lax
from jax.experimental import pallas as pl
from jax.experimental.pallas import tpu as pltpu
from jax.experimental.pallas import tpu_sc as plsc

N = 100000
H = 64
G = 64

NC = 2
NS = 16
NW = NC * NS

NP = 100352
ROWS_PER_TILE = 784
EROWS = ROWS_PER_TILE * NW
EPAD = EROWS * 128
DUMMY = 100224

B = 1024
GRID = NP // B
RPW = NP // NS
K = 4


def _sc_body(num_tables, gather, *refs):
    i = 0
    if gather:
        srcR = refs[i]; i += 1
    dstR = refs[i]; i += 1
    tables = refs[i:i + (num_tables if gather else 0)]
    i += len(tables)
    outs = refs[i:i + num_tables]; i += num_tables
    acc = refs[i]; i += 1
    if gather:
        srcA = refs[i]; srcB = refs[i + 1]; i += 2
    dstA = refs[i]; dstB = refs[i + 1]; i += 2
    if gather:
        rowsA = refs[i]; rowsB = refs[i + 1]; i += 2
    else:
        rowsA = rowsB = refs[i]; i += 1
    zbuf = refs[i]; i += 1
    semgA = refs[i]; semgB = refs[i + 1]; i += 2
    semsA = refs[i]; semsB = refs[i + 1]; i += 2

    core = lax.axis_index("c")
    sub = lax.axis_index("s")
    w = core * NS + sub
    r0 = sub * RPW
    row0 = w * ROWS_PER_TILE

    zero16 = jnp.zeros((16,), jnp.float32)
    for z in range(64):
        zbuf[z, :] = zero16
    if not gather:
        one16 = jnp.ones((16,), jnp.float32)
        for z in range(128):
            rowsA[z, :] = one16

    def zero_acc():
        def zl(k, carry):
            pltpu.sync_copy(zbuf, acc.at[pl.ds(r0 + k * 64, 64)])
            return carry
        lax.fori_loop(0, RPW // 64, zl, 0)

    def _rowref(rows, j):
        return rows.at[j] if gather else rows

    def edge_pass(table):
        def half(t, base, src_buf, dst_buf, rows, semg, sems):
            @pl.when(t > 0)
            def _():
                for j in range(K):
                    pltpu.make_async_copy(
                        _rowref(rows, j), acc.at[dst_buf.at[j]], sems).wait()
            pltpu.sync_copy(dstR.at[pl.ds(base, K)], dst_buf)
            if gather:
                pltpu.sync_copy(srcR.at[pl.ds(base, K)], src_buf)
                for j in range(K):
                    pltpu.async_copy(table.at[src_buf.at[j]], rows.at[j],
                                     semg)

        def fire_scatter(src_buf, dst_buf, rows, semg, sems):
            if gather:
                for j in range(K):
                    pltpu.make_async_copy(table.at[src_buf.at[j]],
                                          rows.at[j], semg).wait()
            for j in range(K):
                pltpu.async_copy(_rowref(rows, j), acc.at[dst_buf.at[j]],
                                 sems, add=True)

        sA = (srcA if gather else None)
        sB = (srcB if gather else None)

        def it(t, carry):
            base = row0 + t * 2 * K
            half(t, base, sA, dstA, rowsA, semgA, semsA)
            half(t, base + K, sB, dstB, rowsB, semgB, semsB)
            fire_scatter(sA, dstA, rowsA, semgA, semsA)
            fire_scatter(sB, dstB, rowsB, semgB, semsB)
            return carry
        lax.fori_loop(0, ROWS_PER_TILE // (2 * K), it, 0)
        for j in range(K):
            pltpu.make_async_copy(
                _rowref(rowsA, j), acc.at[dstA.at[j]], semsA).wait()
            pltpu.make_async_copy(
                _rowref(rowsB, j), acc.at[dstB.at[j]], semsB).wait()

    for p in range(num_tables):
        zero_acc()
        plsc.subcore_barrier()
        edge_pass(tables[p] if gather else None)
        plsc.subcore_barrier()
        pltpu.sync_copy(acc.at[pl.ds(r0, RPW)],
                        outs[p].at[core, pl.ds(r0, RPW)])
        if p + 1 < num_tables:
            plsc.subcore_barrier()


def _make_sc(num_tables, gather):
    mesh = plsc.VectorSubcoreMesh(core_axis_name="c", subcore_axis_name="s",
                                  num_cores=NC, num_subcores=NS)
    out_type = [jax.ShapeDtypeStruct((NC, NP, 16), jnp.float32)
                for _ in range(num_tables)]
    scratch = [pltpu.VMEM_SHARED((NP, 16), jnp.float32)]
    if gather:
        scratch += [pltpu.VMEM((K, 128), jnp.int32)] * 2
    scratch += [pltpu.VMEM((K, 128), jnp.int32)] * 2
    if gather:
        scratch += [pltpu.VMEM((K, 128, 16), jnp.float32)] * 2
    else:
        scratch += [pltpu.VMEM((128, 16), jnp.float32)]
    scratch += [
        pltpu.VMEM((64, 16), jnp.float32),
        pltpu.SemaphoreType.DMA,
        pltpu.SemaphoreType.DMA,
        pltpu.SemaphoreType.DMA,
        pltpu.SemaphoreType.DMA,
    ]
    body = functools.partial(_sc_body, num_tables, gather)
    return pl.kernel(body, out_type=out_type, mesh=mesh,
                     scratch_types=scratch,
                     compiler_params=pltpu.CompilerParams(
                         use_tc_tiling_on_sc=False))


_sc_deg = _make_sc(1, gather=False)
_sc_prop1 = _make_sc(1, gather=True)
_sc_prop4 = _make_sc(4, gather=True)


def _rows_mask(pid):
    idx = pid * B + lax.broadcasted_iota(jnp.int32, (B, 1), 0)
    return idx < N


def _tc1_body(deg_ref, x_ref, dinv_ref, xs_ref):
    pid = pl.program_id(0)
    deg = deg_ref[0, :, 0:1] + deg_ref[1, :, 0:1] + 1.0
    dinv = jnp.where(_rows_mask(pid), lax.rsqrt(deg), 0.0)
    dinv_ref[...] = dinv
    xs_ref[...] = jnp.concatenate(
        [dinv * x_ref[...], jnp.zeros((B, 15), jnp.float32)], axis=1)


def _tc1(deg16, xP):
    return pl.pallas_call(
        _tc1_body,
        grid=(GRID,),
        in_specs=[
            pl.BlockSpec((NC, B, 16), lambda i: (0, i, 0)),
            pl.BlockSpec((B, 1), lambda i: (i, 0)),
        ],
        out_specs=[
            pl.BlockSpec((B, 1), lambda i: (i, 0)),
            pl.BlockSpec((B, 16), lambda i: (i, 0)),
        ],
        out_shape=[
            jax.ShapeDtypeStruct((NP, 1), jnp.float32),
            jax.ShapeDtypeStruct((NP, 16), jnp.float32),
        ],
    )(deg16, xP)


def _tc2_body(p16_ref, xs_ref, dinv_ref, w1_ref, b1_ref, *g_refs):
    dinv = dinv_ref[...]
    p = dinv * (p16_ref[0, :, 0:1] + p16_ref[1, :, 0:1] + xs_ref[:, 0:1])
    h = jax.nn.relu(p * w1_ref[...] + b1_ref[...])
    g = dinv * h
    for k in range(4):
        g_refs[k][...] = g[:, 16 * k:16 * (k + 1)]


def _tc2(p16, xs16, dinv, W1, b1):
    return pl.pallas_call(
        _tc2_body,
        grid=(GRID,),
        in_specs=[
            pl.BlockSpec((NC, B, 16), lambda i: (0, i, 0)),
            pl.BlockSpec((B, 16), lambda i: (i, 0)),
            pl.BlockSpec((B, 1), lambda i: (i, 0)),
            pl.BlockSpec((1, H), lambda i: (0, 0)),
            pl.BlockSpec((1, H), lambda i: (0, 0)),
        ],
        out_specs=[pl.BlockSpec((B, 16), lambda i: (i, 0))] * 4,
        out_shape=[jax.ShapeDtypeStruct((NP, 16), jnp.float32)] * 4,
    )(p16, xs16, dinv, W1, b1)


def _tc3_body(a0, a1, a2, a3, g0, g1, g2, g3, dinv_ref, w_ref, b_ref,
              *out_refs):
    dinv = dinv_ref[...]
    accs = (a0, a1, a2, a3)
    gins = (g0, g1, g2, g3)
    z = jnp.concatenate(
        [accs[k][0] + accs[k][1] + gins[k][...] for k in range(4)], axis=1)
    h = jax.nn.relu(jnp.dot(dinv * z, w_ref[...],
                            preferred_element_type=jnp.float32) + b_ref[...])
    g = dinv * h
    for k in range(4):
        out_refs[k][...] = g[:, 16 * k:16 * (k + 1)]


def _tc3(accs, gins, dinv, W, b):
    return pl.pallas_call(
        _tc3_body,
        grid=(GRID,),
        in_specs=(
            [pl.BlockSpec((NC, B, 16), lambda i: (0, i, 0))] * 4
            + [pl.BlockSpec((B, 16), lambda i: (i, 0))] * 4
            + [
                pl.BlockSpec((B, 1), lambda i: (i, 0)),
                pl.BlockSpec((H, H), lambda i: (0, 0)),
                pl.BlockSpec((1, H), lambda i: (0, 0)),
            ]
        ),
        out_specs=[pl.BlockSpec((B, 16), lambda i: (i, 0))] * 4,
        out_shape=[jax.ShapeDtypeStruct((NP, 16), jnp.float32)] * 4,
    )(*accs, *gins, dinv, W, b)


def _tc4_body(a0, a1, a2, a3, g0, g1, g2, g3, dinv_ref, batch_ref, w3_ref,
              b3_ref, wfc_ref, bfc_ref, out_ref, s_acc, c_acc):
    pid = pl.program_id(0)
    dinv = dinv_ref[...]
    accs = (a0, a1, a2, a3)
    gins = (g0, g1, g2, g3)
    z = jnp.concatenate(
        [accs[k][0] + accs[k][1] + gins[k][...] for k in range(4)], axis=1)
    h = jax.nn.relu(jnp.dot(dinv * z, w3_ref[...],
                            preferred_element_type=jnp.float32) + b3_ref[...])
    seg = batch_ref[...]
    segs = lax.broadcasted_iota(jnp.int32, (B, G), 1)
    onehot = jnp.where((seg == segs) & _rows_mask(pid), 1.0, 0.0)

    s_part = lax.dot_general(onehot, h, (((0,), (0,)), ((), ())),
                             preferred_element_type=jnp.float32)
    c_part = lax.dot_general(onehot, jnp.ones((B, 1), jnp.float32),
                             (((0,), (0,)), ((), ())),
                             preferred_element_type=jnp.float32)

    @pl.when(pid == 0)
    def _():
        s_acc[...] = jnp.zeros_like(s_acc)
        c_acc[...] = jnp.zeros_like(c_acc)

    s_acc[...] += s_part
    c_acc[...] += c_part

    @pl.when(pid == GRID - 1)
    def _():
        pooled = s_acc[...] / jnp.maximum(c_acc[...], 1.0)
        out_ref[...] = jnp.dot(pooled, wfc_ref[...],
                               preferred_element_type=jnp.float32) + bfc_ref[...]


def _tc4(accs, gins, dinv, batchP, W3, b3, Wfc, bfc):
    return pl.pallas_call(
        _tc4_body,
        grid=(GRID,),
        in_specs=(
            [pl.BlockSpec((NC, B, 16), lambda i: (0, i, 0))] * 4
            + [pl.BlockSpec((B, 16), lambda i: (i, 0))] * 4
            + [
                pl.BlockSpec((B, 1), lambda i: (i, 0)),
                pl.BlockSpec((B, 1), lambda i: (i, 0)),
                pl.BlockSpec((H, H), lambda i: (0, 0)),
                pl.BlockSpec((1, H), lambda i: (0, 0)),
                pl.BlockSpec((H, 1), lambda i: (0, 0)),
                pl.BlockSpec((1, 1), lambda i: (0, 0)),
            ]
        ),
        out_specs=pl.BlockSpec((G, 1), lambda i: (0, 0)),
        out_shape=jax.ShapeDtypeStruct((G, 1), jnp.float32),
        scratch_shapes=[
            pltpu.VMEM((G, H), jnp.float32),
            pltpu.VMEM((G, 1), jnp.float32),
        ],
    )(*accs, *gins, dinv, batchP, W3, b3, Wfc, bfc)


def kernel(x, edge_index, batch, W1, b1, W2, b2, W3, b3, Wfc, bfc):
    E = edge_index.shape[1]
    pad_e = EPAD - E
    fill = jnp.full((pad_e,), DUMMY, jnp.int32)
    srcR = jnp.concatenate([edge_index[0], fill]).reshape(EROWS, 128)
    dstR = jnp.concatenate([edge_index[1], fill]).reshape(EROWS, 128)

    xP = jnp.pad(x, ((0, NP - N), (0, 0)))
    batchP = jnp.pad(batch, (0, NP - N)).reshape(NP, 1)
    b1r = b1.reshape(1, H)
    b2r = b2.reshape(1, H)
    b3r = b3.reshape(1, H)
    bfcr = bfc.reshape(1, 1)

    (deg16,) = _sc_deg(dstR)
    dinv, xs16 = _tc1(deg16, xP)
    (p16,) = _sc_prop1(srcR, dstR, xs16)
    g1 = _tc2(p16, xs16, dinv, W1, b1r)
    a2 = _sc_prop4(srcR, dstR, *g1)
    g2 = _tc3(a2, g1, dinv, W2, b2r)
    a3 = _sc_prop4(srcR, dstR, *g2)
    return _tc4(a3, g2, dinv, batchP, W3, b3r, Wfc, bfcr)
